# baseline (speedup 1.0000x reference)
import functools

import jax
import jax.numpy as jnp
from jax import lax
from jax.experimental import pallas as pl
from jax.experimental.pallas import tpu as pltpu

N = 32768
K = 32
R0 = 0.2
R1 = 0.5
M1 = N // K
M2 = M1 // K
D0 = 64
D1 = 128


def _fps_kernel(xs_ref, ys_ref, zs_ref, pr_ref, sel_ref, p1_ref, mind_ref, *, m, S, L):
    idx_plane = (lax.broadcasted_iota(jnp.int32, (S, L), 0) * L
                 + lax.broadcasted_iota(jnp.int32, (S, L), 1))
    big = jnp.int32(S * L)

    row0 = pr_ref[0:1, :]
    qx = row0[:, 0:1]
    qy = row0[:, 1:2]
    qz = row0[:, 2:3]
    mind_ref[...] = ((xs_ref[...] - qx) ** 2 + (ys_ref[...] - qy) ** 2
                     + (zs_ref[...] - qz) ** 2)
    sel_ref[0] = jnp.int32(0)
    p1_ref[0:1, :] = row0

    def body(i, _):
        mind = mind_ref[...]
        mx = jnp.max(mind)
        nxt = jnp.min(jnp.where(mind == mx, idx_plane, big))
        sel_ref[i] = nxt
        row = pr_ref[pl.ds(nxt, 1), :]
        p1_ref[pl.ds(i, 1), :] = row
        qx = row[:, 0:1]
        qy = row[:, 1:2]
        qz = row[:, 2:3]
        d2 = ((xs_ref[...] - qx) ** 2 + (ys_ref[...] - qy) ** 2
              + (zs_ref[...] - qz) ** 2)
        mind_ref[...] = jnp.minimum(mind, d2)
        return 0

    lax.fori_loop(1, m, body, 0)


def _fps(pts, m, S, L):
    n = pts.shape[0]
    planes = pts.T.reshape(3, S, L)
    sel, p_sel = pl.pallas_call(
        functools.partial(_fps_kernel, m=m, S=S, L=L),
        out_shape=(
            jax.ShapeDtypeStruct((m,), jnp.int32),
            jax.ShapeDtypeStruct((m, 3), jnp.float32),
        ),
        in_specs=[
            pl.BlockSpec(memory_space=pltpu.MemorySpace.VMEM),
            pl.BlockSpec(memory_space=pltpu.MemorySpace.VMEM),
            pl.BlockSpec(memory_space=pltpu.MemorySpace.VMEM),
            pl.BlockSpec(memory_space=pltpu.MemorySpace.VMEM),
        ],
        out_specs=(
            pl.BlockSpec(memory_space=pltpu.MemorySpace.SMEM),
            pl.BlockSpec(memory_space=pltpu.MemorySpace.VMEM),
        ),
        scratch_shapes=[pltpu.VMEM((S, L), jnp.float32)],
    )(planes[0], planes[1], planes[2], pts)
    return sel, p_sel


def _d2_block_kernel(y_ref, xt_ref, out_ref):
    y = y_ref[...]
    xt = xt_ref[...]
    sy = jnp.sum(y * y, axis=1, keepdims=True)
    sx = jnp.sum(xt * xt, axis=0, keepdims=True)
    out_ref[...] = sy + sx - 2.0 * jnp.dot(y, xt)


def _d2_matrix(y, x, blk):
    m = y.shape[0]
    n = x.shape[0]
    xt = x.T
    return pl.pallas_call(
        _d2_block_kernel,
        grid=(n // blk,),
        in_specs=[
            pl.BlockSpec((m, 3), lambda i: (0, 0)),
            pl.BlockSpec((3, blk), lambda i: (0, i)),
        ],
        out_specs=pl.BlockSpec((m, blk), lambda i: (0, i)),
        out_shape=jax.ShapeDtypeStruct((m, n), jnp.float32),
    )(y, xt)


_SC_NC = 2
_SC_NS = 16
_SC_NW = _SC_NC * _SC_NS

_BUF = 160
_TRIG = 128
_GRP = 8


def _sc_topk_body(d2_ref, idx_ref, d2o_ref, row_v, bufd, bufi, outd, outi,
                  tmpf, tmpi, cnt_s, t_s, *, m, n, k):
    import jax.experimental.pallas.tpu_sc as plsc

    rows_per = m // _SC_NW
    nb = _BUF // 16
    n_groups = n // (16 * _GRP)
    wid = lax.axis_index("s") * _SC_NC + lax.axis_index("c")
    inf = jnp.float32(jnp.inf)
    bigi = jnp.int32(2**30)
    lane = lax.iota(jnp.int32, 16)
    inf_v = jnp.full((16,), inf, jnp.float32)
    bigi_v = jnp.full((16,), bigi, jnp.int32)

    def _smin_f(vec):
        for d in (8, 4, 2, 1):
            tmpf[0:16] = vec
            vec = jnp.minimum(vec, plsc.load_gather(tmpf, [lane ^ d]))
        return vec[0]

    def _smin_i(vec):
        for d in (8, 4, 2, 1):
            tmpi[0:16] = vec
            vec = jnp.minimum(vec, plsc.load_gather(tmpi, [lane ^ d]))
        return vec[0]

    def _ssum_i(vec):
        for d in (8, 4, 2, 1):
            tmpi[0:16] = vec
            vec = vec + plsc.load_gather(tmpi, [lane ^ d])
        return vec[0]

    def distill():
        def extract(t, st):
            o0d, o1d, o0i, o1i, _ = st
            acc = bufd[0:16]
            for b in range(1, nb):
                acc = jnp.minimum(acc, bufd[pl.ds(b * 16, 16)])
            mv = _smin_f(acc)
            ii = bigi_v
            for b in range(nb):
                v = bufd[pl.ds(b * 16, 16)]
                ix = bufi[pl.ds(b * 16, 16)]
                ii = jnp.minimum(ii, jnp.where(v == mv, ix, bigi))
            mi = _smin_i(ii)
            for b in range(nb):
                v = bufd[pl.ds(b * 16, 16)]
                ix = bufi[pl.ds(b * 16, 16)]
                w = (v == mv) & (ix == mi)
                bufd[pl.ds(b * 16, 16)] = jnp.where(w, inf, v)
            tl = jnp.where(t < 16, t, t - 16)
            le = lane == tl
            in0 = t < 16
            o0d = jnp.where(le & in0, mv, o0d)
            o1d = jnp.where(le & (~in0), mv, o1d)
            o0i = jnp.where(le & in0, mi, o0i)
            o1i = jnp.where(le & (~in0), mi, o1i)
            return (o0d, o1d, o0i, o1i, mv)

        o0d, o1d, o0i, o1i, lmv = lax.fori_loop(
            0, k, extract, (inf_v, inf_v, bigi_v, bigi_v, inf))
        outd[0:16] = o0d
        outd[pl.ds(16, 16)] = o1d
        outi[0:16] = o0i
        outi[pl.ds(16, 16)] = o1i
        bufd[0:16] = o0d
        bufd[pl.ds(16, 16)] = o1d
        bufi[0:16] = o0i
        bufi[pl.ds(16, 16)] = o1i
        for b in range(2, nb):
            bufd[pl.ds(b * 16, 16)] = inf_v
            bufi[pl.ds(b * 16, 16)] = bigi_v
        cnt_s[0] = jnp.int32(k)
        t_s[0] = lmv

    def do_row(j, _):
        row = wid * rows_per + j
        pltpu.sync_copy(d2_ref.at[row], row_v)
        for b in range(nb):
            bufd[pl.ds(b * 16, 16)] = inf_v
            bufi[pl.ds(b * 16, 16)] = bigi_v
        cnt_s[0] = jnp.int32(0)
        t_s[0] = inf

        def group(g, _):
            base = g * (16 * _GRP)
            acc = row_v[pl.ds(base, 16)]
            for b in range(1, _GRP):
                acc = jnp.minimum(acc, row_v[pl.ds(base + b * 16, 16)])
            gm = _smin_f(acc)

            @pl.when(gm < t_s[0])
            def _():
                for b in range(_GRP):
                    v = row_v[pl.ds(base + b * 16, 16)]
                    msk = v < t_s[0]
                    nn = _ssum_i(msk.astype(jnp.int32))

                    @pl.when(nn > 0)
                    def _():
                        c = cnt_s[0]
                        plsc.store_compressed(bufd.at[pl.ds(c, 16)], v, mask=msk)
                        plsc.store_compressed(
                            bufi.at[pl.ds(c, 16)], base + b * 16 + lane, mask=msk)
                        cnt_s[0] = c + nn

                        @pl.when(c + nn >= _TRIG)
                        def _():
                            distill()

            return 0

        lax.fori_loop(0, n_groups, group, 0)
        distill()
        pltpu.sync_copy(outi, idx_ref.at[row])
        pltpu.sync_copy(outd, d2o_ref.at[row])
        return 0

    lax.fori_loop(0, rows_per, do_row, 0)


def _sc_topk(d2, k):
    import jax.experimental.pallas.tpu_sc as plsc

    m, n = d2.shape
    mesh = plsc.VectorSubcoreMesh(core_axis_name="c", subcore_axis_name="s")
    fn = pl.kernel(
        functools.partial(_sc_topk_body, m=m, n=n, k=k),
        out_type=(
            jax.ShapeDtypeStruct((m, k), jnp.int32),
            jax.ShapeDtypeStruct((m, k), jnp.float32),
        ),
        mesh=mesh,
        compiler_params=pltpu.CompilerParams(needs_layout_passes=False),
        scratch_types=[
            pltpu.VMEM((n,), jnp.float32),
            pltpu.VMEM((_BUF,), jnp.float32),
            pltpu.VMEM((_BUF,), jnp.int32),
            pltpu.VMEM((k,), jnp.float32),
            pltpu.VMEM((k,), jnp.int32),
            pltpu.VMEM((16,), jnp.float32),
            pltpu.VMEM((16,), jnp.int32),
            pltpu.SMEM((1,), jnp.int32),
            pltpu.SMEM((1,), jnp.float32),
        ],
    )
    return fn(d2)


def _knn_radius(x, y, r, k, blk):
    d2 = _d2_matrix(y, x, blk)
    idx, d2sel = _sc_topk(d2, k)
    valid = d2sel <= r * r
    return idx, valid


def kernel(points, batch, enc0_W, enc0_b, enc1_W, enc1_b, dec0_W, dec0_b, dec1_W, dec1_b):
    del batch
    fps1, p1 = _fps(points, M1, 8, N // 8)
    fps2, p2 = _fps(p1, M2, 8, M1 // 8)

    idx0, valid0 = _knn_radius(points, p1, R0, K, 2048)
    rel0 = jnp.where(valid0[..., None], (points[idx0] - p1[:, None, :]) / R0, 0.0)
    h0 = jax.nn.relu(rel0.reshape(-1, 3) @ enc0_W + enc0_b)
    h0 = jnp.where(valid0.reshape(-1, 1), h0, 0.0)
    f1 = h0.reshape(M1, K, D0).max(axis=1)

    idx1, valid1 = _knn_radius(p1, p2, R1, K, 1024)
    rel1 = jnp.where(valid1[..., None], (p1[idx1] - p2[:, None, :]) / R1, 0.0)
    g1 = jnp.where(valid1[..., None], f1[idx1], 0.0)
    h1 = jax.nn.relu(jnp.concatenate([rel1, g1], axis=-1).reshape(-1, 3 + D0) @ enc1_W + enc1_b)
    h1 = jnp.where(valid1.reshape(-1, 1), h1, 0.0)
    f2 = h1.reshape(M2, K, D1).max(axis=1)

    cur = idx1.reshape(-1)
    input_points1 = p1[cur]
    nxt = idx0[cur].reshape(-1)
    input_points0 = points[nxt]

    d0 = (f2 @ dec0_W + dec0_b).reshape(M2, K, 3 + D0)
    rel_a = jnp.tanh(d0[..., :3]).reshape(M2 * K, 3)
    feat_a = jax.nn.relu(d0[..., 3:]).reshape(M2 * K, D0)
    d1 = (feat_a @ dec1_W + dec1_b).reshape(M2 * K, K, 3)
    rel_b = jnp.tanh(d1)
    out1 = p2
    out2 = (out1[:, None, :] + rel_a.reshape(M2, K, 3) * R1).reshape(M2 * K, 3)
    out3 = (out2[:, None, :] + rel_b * R0).reshape(M2 * K * K, 3)
    return (out3, f2, input_points0, input_points1)


# speedup vs baseline: 1.3832x; 1.3832x over previous
import functools

import jax
import jax.numpy as jnp
from jax import lax
from jax.experimental import pallas as pl
from jax.experimental.pallas import tpu as pltpu

N = 32768
K = 32
R0 = 0.2
R1 = 0.5
M1 = N // K
M2 = M1 // K
D0 = 64
D1 = 128


def _fps_kernel(xs_ref, ys_ref, zs_ref, pr_ref, sel_ref, p1_ref, mind_ref, *, m, S, L):
    idx_plane = (lax.broadcasted_iota(jnp.int32, (S, L), 0) * L
                 + lax.broadcasted_iota(jnp.int32, (S, L), 1))
    big = jnp.int32(S * L)

    row0 = pr_ref[0:1, :]
    qx = row0[:, 0:1]
    qy = row0[:, 1:2]
    qz = row0[:, 2:3]
    mind_ref[...] = ((xs_ref[...] - qx) ** 2 + (ys_ref[...] - qy) ** 2
                     + (zs_ref[...] - qz) ** 2)
    sel_ref[0] = jnp.int32(0)
    p1_ref[0:1, :] = row0

    def body(i, _):
        mind = mind_ref[...]
        mx = jnp.max(mind)
        nxt = jnp.min(jnp.where(mind == mx, idx_plane, big))
        sel_ref[i] = nxt
        row = pr_ref[pl.ds(nxt, 1), :]
        p1_ref[pl.ds(i, 1), :] = row
        qx = row[:, 0:1]
        qy = row[:, 1:2]
        qz = row[:, 2:3]
        d2 = ((xs_ref[...] - qx) ** 2 + (ys_ref[...] - qy) ** 2
              + (zs_ref[...] - qz) ** 2)
        mind_ref[...] = jnp.minimum(mind, d2)
        return 0

    lax.fori_loop(1, m, body, 0)


def _fps(pts, m, S, L):
    n = pts.shape[0]
    planes = pts.T.reshape(3, S, L)
    sel, p_sel = pl.pallas_call(
        functools.partial(_fps_kernel, m=m, S=S, L=L),
        out_shape=(
            jax.ShapeDtypeStruct((m,), jnp.int32),
            jax.ShapeDtypeStruct((m, 3), jnp.float32),
        ),
        in_specs=[
            pl.BlockSpec(memory_space=pltpu.MemorySpace.VMEM),
            pl.BlockSpec(memory_space=pltpu.MemorySpace.VMEM),
            pl.BlockSpec(memory_space=pltpu.MemorySpace.VMEM),
            pl.BlockSpec(memory_space=pltpu.MemorySpace.VMEM),
        ],
        out_specs=(
            pl.BlockSpec(memory_space=pltpu.MemorySpace.SMEM),
            pl.BlockSpec(memory_space=pltpu.MemorySpace.VMEM),
        ),
        scratch_shapes=[pltpu.VMEM((S, L), jnp.float32)],
    )(planes[0], planes[1], planes[2], pts)
    return sel, p_sel


def _d2_block_kernel(y_ref, xt_ref, out_ref, gmin_ref, *, gsz):
    y = y_ref[...]
    xt = xt_ref[...]
    sy = jnp.sum(y * y, axis=1, keepdims=True)
    sx = jnp.sum(xt * xt, axis=0, keepdims=True)
    d2 = sy + sx - 2.0 * jnp.dot(y, xt)
    out_ref[...] = d2
    m, blk = d2.shape
    gmin_ref[...] = d2.reshape(m, blk // gsz, gsz).min(axis=-1)[None]


def _d2_matrix(y, x, blk, gsz):
    m = y.shape[0]
    n = x.shape[0]
    xt = x.T
    return pl.pallas_call(
        functools.partial(_d2_block_kernel, gsz=gsz),
        grid=(n // blk,),
        in_specs=[
            pl.BlockSpec((m, 3), lambda i: (0, 0)),
            pl.BlockSpec((3, blk), lambda i: (0, i)),
        ],
        out_specs=(
            pl.BlockSpec((m, blk), lambda i: (0, i)),
            pl.BlockSpec((1, m, blk // gsz), lambda i: (i, 0, 0)),
        ),
        out_shape=(
            jax.ShapeDtypeStruct((m, n), jnp.float32),
            jax.ShapeDtypeStruct((n // blk, m, blk // gsz), jnp.float32),
        ),
    )(y, xt)


_SC_NC = 2
_SC_NS = 16
_SC_NW = _SC_NC * _SC_NS

_BUF = 160
_TRIG = 128
_GRP = 8


def _sc_topk_body(d2_ref, gmin_ref, idx_ref, d2o_ref, row_v, gmin_v,
                  bufd, bufi, outd, outi, tmpf, tmpi, cnt_s, t_s,
                  *, m, n, k, grp):
    import jax.experimental.pallas.tpu_sc as plsc

    rows_per = m // _SC_NW
    nb = _BUF // 16
    gsz = 16 * grp
    n_groups = n // gsz
    wid = lax.axis_index("s") * _SC_NC + lax.axis_index("c")
    inf = jnp.float32(jnp.inf)
    bigi = jnp.int32(2**30)
    lane = lax.iota(jnp.int32, 16)
    inf_v = jnp.full((16,), inf, jnp.float32)
    bigi_v = jnp.full((16,), bigi, jnp.int32)

    def _smin_f(vec):
        for d in (8, 4, 2, 1):
            tmpf[0:16] = vec
            vec = jnp.minimum(vec, plsc.load_gather(tmpf, [lane ^ d]))
        return vec[0]

    def _smin_i(vec):
        for d in (8, 4, 2, 1):
            tmpi[0:16] = vec
            vec = jnp.minimum(vec, plsc.load_gather(tmpi, [lane ^ d]))
        return vec[0]

    def distill():
        def extract(t, st):
            o0d, o1d, o0i, o1i, _ = st
            acc = bufd[0:16]
            for b in range(1, nb):
                acc = jnp.minimum(acc, bufd[pl.ds(b * 16, 16)])
            mv = _smin_f(acc)
            ii = bigi_v
            for b in range(nb):
                v = bufd[pl.ds(b * 16, 16)]
                ix = bufi[pl.ds(b * 16, 16)]
                ii = jnp.minimum(ii, jnp.where(v == mv, ix, bigi))
            mi = _smin_i(ii)
            for b in range(nb):
                v = bufd[pl.ds(b * 16, 16)]
                ix = bufi[pl.ds(b * 16, 16)]
                w = (v == mv) & (ix == mi)
                bufd[pl.ds(b * 16, 16)] = jnp.where(w, inf, v)
            tl = jnp.where(t < 16, t, t - 16)
            le = lane == tl
            in0 = t < 16
            o0d = jnp.where(le & in0, mv, o0d)
            o1d = jnp.where(le & (~in0), mv, o1d)
            o0i = jnp.where(le & in0, mi, o0i)
            o1i = jnp.where(le & (~in0), mi, o1i)
            return (o0d, o1d, o0i, o1i, mv)

        o0d, o1d, o0i, o1i, lmv = lax.fori_loop(
            0, k, extract, (inf_v, inf_v, bigi_v, bigi_v, inf))
        outd[0:16] = o0d
        outd[pl.ds(16, 16)] = o1d
        outi[0:16] = o0i
        outi[pl.ds(16, 16)] = o1i
        bufd[0:16] = o0d
        bufd[pl.ds(16, 16)] = o1d
        bufi[0:16] = o0i
        bufi[pl.ds(16, 16)] = o1i
        for b in range(2, nb):
            bufd[pl.ds(b * 16, 16)] = inf_v
            bufi[pl.ds(b * 16, 16)] = bigi_v
        cnt_s[0] = jnp.int32(k)
        t_s[0] = lmv

    def do_row(j, _):
        row = wid * rows_per + j
        pltpu.sync_copy(d2_ref.at[row], row_v)
        pltpu.sync_copy(gmin_ref.at[row], gmin_v)
        for b in range(nb):
            bufd[pl.ds(b * 16, 16)] = inf_v
            bufi[pl.ds(b * 16, 16)] = bigi_v
        cnt_s[0] = jnp.int32(0)
        t_s[0] = inf

        def group(g, _):
            sg = plsc.load_gather(gmin_v, [jnp.full((16,), g, jnp.int32)])[0]

            @pl.when(sg < t_s[0])
            def _():
                base = g * gsz
                for b in range(grp):
                    v = row_v[pl.ds(base + b * 16, 16)]
                    msk = v < t_s[0]
                    nn = plsc.all_reduce_population_count(msk)[0]

                    @pl.when(nn > 0)
                    def _():
                        c = cnt_s[0]
                        plsc.store_compressed(bufd.at[pl.ds(c, 16)], v, mask=msk)
                        plsc.store_compressed(
                            bufi.at[pl.ds(c, 16)], base + b * 16 + lane, mask=msk)
                        cnt_s[0] = c + nn

                        @pl.when(c + nn >= _TRIG)
                        def _():
                            distill()

            return 0

        lax.fori_loop(0, n_groups, group, 0)
        distill()
        pltpu.sync_copy(outi, idx_ref.at[row])
        pltpu.sync_copy(outd, d2o_ref.at[row])
        return 0

    lax.fori_loop(0, rows_per, do_row, 0)


def _sc_topk(d2, gmin, k, grp):
    import jax.experimental.pallas.tpu_sc as plsc

    m, n = d2.shape
    ng = n // (16 * grp)
    mesh = plsc.VectorSubcoreMesh(core_axis_name="c", subcore_axis_name="s")
    fn = pl.kernel(
        functools.partial(_sc_topk_body, m=m, n=n, k=k, grp=grp),
        out_type=(
            jax.ShapeDtypeStruct((m, k), jnp.int32),
            jax.ShapeDtypeStruct((m, k), jnp.float32),
        ),
        mesh=mesh,
        compiler_params=pltpu.CompilerParams(needs_layout_passes=False),
        scratch_types=[
            pltpu.VMEM((n,), jnp.float32),
            pltpu.VMEM((ng,), jnp.float32),
            pltpu.VMEM((_BUF,), jnp.float32),
            pltpu.VMEM((_BUF,), jnp.int32),
            pltpu.VMEM((k,), jnp.float32),
            pltpu.VMEM((k,), jnp.int32),
            pltpu.VMEM((16,), jnp.float32),
            pltpu.VMEM((16,), jnp.int32),
            pltpu.SMEM((1,), jnp.int32),
            pltpu.SMEM((1,), jnp.float32),
        ],
    )
    return fn(d2, gmin)


def _knn_radius(x, y, r, k, blk, grp):
    d2, gmin3 = _d2_matrix(y, x, blk, 16 * grp)
    gmin = gmin3.transpose(1, 0, 2).reshape(y.shape[0], -1)
    idx, d2sel = _sc_topk(d2, gmin, k, grp)
    valid = d2sel <= r * r
    return idx, valid


def kernel(points, batch, enc0_W, enc0_b, enc1_W, enc1_b, dec0_W, dec0_b, dec1_W, dec1_b):
    del batch
    fps1, p1 = _fps(points, M1, 8, N // 8)
    fps2, p2 = _fps(p1, M2, 8, M1 // 8)

    idx0, valid0 = _knn_radius(points, p1, R0, K, 2048, 8)
    rel0 = jnp.where(valid0[..., None], (points[idx0] - p1[:, None, :]) / R0, 0.0)
    h0 = jax.nn.relu(rel0.reshape(-1, 3) @ enc0_W + enc0_b)
    h0 = jnp.where(valid0.reshape(-1, 1), h0, 0.0)
    f1 = h0.reshape(M1, K, D0).max(axis=1)

    idx1, valid1 = _knn_radius(p1, p2, R1, K, 1024, 4)
    rel1 = jnp.where(valid1[..., None], (p1[idx1] - p2[:, None, :]) / R1, 0.0)
    g1 = jnp.where(valid1[..., None], f1[idx1], 0.0)
    h1 = jax.nn.relu(jnp.concatenate([rel1, g1], axis=-1).reshape(-1, 3 + D0) @ enc1_W + enc1_b)
    h1 = jnp.where(valid1.reshape(-1, 1), h1, 0.0)
    f2 = h1.reshape(M2, K, D1).max(axis=1)

    cur = idx1.reshape(-1)
    input_points1 = p1[cur]
    nxt = idx0[cur].reshape(-1)
    input_points0 = points[nxt]

    d0 = (f2 @ dec0_W + dec0_b).reshape(M2, K, 3 + D0)
    rel_a = jnp.tanh(d0[..., :3]).reshape(M2 * K, 3)
    feat_a = jax.nn.relu(d0[..., 3:]).reshape(M2 * K, D0)
    d1 = (feat_a @ dec1_W + dec1_b).reshape(M2 * K, K, 3)
    rel_b = jnp.tanh(d1)
    out1 = p2
    out2 = (out1[:, None, :] + rel_a.reshape(M2, K, 3) * R1).reshape(M2 * K, 3)
    out3 = (out2[:, None, :] + rel_b * R0).reshape(M2 * K * K, 3)
    return (out3, f2, input_points0, input_points1)


# speedup vs baseline: 2.4570x; 1.7763x over previous
import functools

import jax
import jax.numpy as jnp
from jax import lax
from jax.experimental import pallas as pl
from jax.experimental.pallas import tpu as pltpu

N = 32768
K = 32
R0 = 0.2
R1 = 0.5
M1 = N // K
M2 = M1 // K
D0 = 64
D1 = 128


def _fps_kernel(xs_ref, ys_ref, zs_ref, pr_ref, sel_ref, p1_ref, mind_ref, *, m, S, L):
    idx_plane = (lax.broadcasted_iota(jnp.int32, (S, L), 0) * L
                 + lax.broadcasted_iota(jnp.int32, (S, L), 1))
    big = jnp.int32(S * L)

    row0 = pr_ref[0:1, :]
    qx = row0[:, 0:1]
    qy = row0[:, 1:2]
    qz = row0[:, 2:3]
    mind_ref[...] = ((xs_ref[...] - qx) ** 2 + (ys_ref[...] - qy) ** 2
                     + (zs_ref[...] - qz) ** 2)
    sel_ref[0] = jnp.int32(0)
    p1_ref[0:1, :] = row0

    def body(i, _):
        mind = mind_ref[...]
        mx = jnp.max(mind)
        nxt = jnp.min(jnp.where(mind == mx, idx_plane, big))
        sel_ref[i] = nxt
        row = pr_ref[pl.ds(nxt, 1), :]
        p1_ref[pl.ds(i, 1), :] = row
        qx = row[:, 0:1]
        qy = row[:, 1:2]
        qz = row[:, 2:3]
        d2 = ((xs_ref[...] - qx) ** 2 + (ys_ref[...] - qy) ** 2
              + (zs_ref[...] - qz) ** 2)
        mind_ref[...] = jnp.minimum(mind, d2)
        return 0

    lax.fori_loop(1, m, body, 0)


def _fps(pts, m, S, L):
    n = pts.shape[0]
    planes = pts.T.reshape(3, S, L)
    sel, p_sel = pl.pallas_call(
        functools.partial(_fps_kernel, m=m, S=S, L=L),
        out_shape=(
            jax.ShapeDtypeStruct((m,), jnp.int32),
            jax.ShapeDtypeStruct((m, 3), jnp.float32),
        ),
        in_specs=[
            pl.BlockSpec(memory_space=pltpu.MemorySpace.VMEM),
            pl.BlockSpec(memory_space=pltpu.MemorySpace.VMEM),
            pl.BlockSpec(memory_space=pltpu.MemorySpace.VMEM),
            pl.BlockSpec(memory_space=pltpu.MemorySpace.VMEM),
        ],
        out_specs=(
            pl.BlockSpec(memory_space=pltpu.MemorySpace.SMEM),
            pl.BlockSpec(memory_space=pltpu.MemorySpace.VMEM),
        ),
        scratch_shapes=[pltpu.VMEM((S, L), jnp.float32)],
    )(planes[0], planes[1], planes[2], pts)
    return sel, p_sel


def _d2_block_kernel(y_ref, xt_ref, out_ref, gmin_ref, *, gsz):
    y = y_ref[...]
    xt = xt_ref[...]
    sy = jnp.sum(y * y, axis=1, keepdims=True)
    sx = jnp.sum(xt * xt, axis=0, keepdims=True)
    d2 = sy + sx - 2.0 * jnp.dot(y, xt)
    out_ref[...] = d2
    m, blk = d2.shape
    gmin_ref[...] = d2.reshape(m, blk // gsz, gsz).min(axis=-1)[None]


def _d2_matrix(y, x, blk, gsz):
    m = y.shape[0]
    n = x.shape[0]
    xt = x.T
    return pl.pallas_call(
        functools.partial(_d2_block_kernel, gsz=gsz),
        grid=(n // blk,),
        in_specs=[
            pl.BlockSpec((m, 3), lambda i: (0, 0)),
            pl.BlockSpec((3, blk), lambda i: (0, i)),
        ],
        out_specs=(
            pl.BlockSpec((m, blk), lambda i: (0, i)),
            pl.BlockSpec((1, m, blk // gsz), lambda i: (i, 0, 0)),
        ),
        out_shape=(
            jax.ShapeDtypeStruct((m, n), jnp.float32),
            jax.ShapeDtypeStruct((n // blk, m, blk // gsz), jnp.float32),
        ),
    )(y, xt)


_SC_NC = 2
_SC_NS = 16
_SC_NW = _SC_NC * _SC_NS

_BUF = 160
_TRIG = 128
_GRP = 8


def _sc_topk_body(d2_ref, gmin_ref, idx_ref, d2o_ref, row_v, gmin_v, ghits,
                  bufd, bufi, outd, outi, tmpf, tmpi, cnt_s, t_s,
                  *, m, n, k, grp):
    import jax.experimental.pallas.tpu_sc as plsc

    rows_per = m // _SC_NW
    nb = _BUF // 16
    gsz = 16 * grp
    n_groups = n // gsz
    wid = lax.axis_index("s") * _SC_NC + lax.axis_index("c")
    inf = jnp.float32(jnp.inf)
    bigi = jnp.int32(2**30)
    lane = lax.iota(jnp.int32, 16)
    inf_v = jnp.full((16,), inf, jnp.float32)
    bigi_v = jnp.full((16,), bigi, jnp.int32)

    def _smin_f(vec):
        for d in (8, 4, 2, 1):
            tmpf[0:16] = vec
            vec = jnp.minimum(vec, plsc.load_gather(tmpf, [lane ^ d]))
        return vec[0]

    def _smin_i(vec):
        for d in (8, 4, 2, 1):
            tmpi[0:16] = vec
            vec = jnp.minimum(vec, plsc.load_gather(tmpi, [lane ^ d]))
        return vec[0]

    def distill():
        def extract(t, st):
            o0d, o1d, o0i, o1i, _ = st
            acc = bufd[0:16]
            for b in range(1, nb):
                acc = jnp.minimum(acc, bufd[pl.ds(b * 16, 16)])
            mv = _smin_f(acc)
            ii = bigi_v
            for b in range(nb):
                v = bufd[pl.ds(b * 16, 16)]
                ix = bufi[pl.ds(b * 16, 16)]
                ii = jnp.minimum(ii, jnp.where(v == mv, ix, bigi))
            mi = _smin_i(ii)
            for b in range(nb):
                v = bufd[pl.ds(b * 16, 16)]
                ix = bufi[pl.ds(b * 16, 16)]
                w = (v == mv) & (ix == mi)
                bufd[pl.ds(b * 16, 16)] = jnp.where(w, inf, v)
            tl = jnp.where(t < 16, t, t - 16)
            le = lane == tl
            in0 = t < 16
            o0d = jnp.where(le & in0, mv, o0d)
            o1d = jnp.where(le & (~in0), mv, o1d)
            o0i = jnp.where(le & in0, mi, o0i)
            o1i = jnp.where(le & (~in0), mi, o1i)
            return (o0d, o1d, o0i, o1i, mv)

        o0d, o1d, o0i, o1i, lmv = lax.fori_loop(
            0, k, extract, (inf_v, inf_v, bigi_v, bigi_v, inf))
        outd[0:16] = o0d
        outd[pl.ds(16, 16)] = o1d
        outi[0:16] = o0i
        outi[pl.ds(16, 16)] = o1i
        bufd[0:16] = o0d
        bufd[pl.ds(16, 16)] = o1d
        bufi[0:16] = o0i
        bufi[pl.ds(16, 16)] = o1i
        for b in range(2, nb):
            bufd[pl.ds(b * 16, 16)] = inf_v
            bufi[pl.ds(b * 16, 16)] = bigi_v
        cnt_s[0] = jnp.int32(k)
        t_s[0] = lmv

    def do_row(j, _):
        row = wid * rows_per + j
        pltpu.sync_copy(d2_ref.at[row], row_v)
        pltpu.sync_copy(gmin_ref.at[row], gmin_v)
        for b in range(nb):
            bufd[pl.ds(b * 16, 16)] = inf_v
            bufi[pl.ds(b * 16, 16)] = bigi_v
        cnt_s[0] = jnp.int32(0)

        def bs(i, st):
            lo, hi = st
            mid = 0.5 * (lo + hi)
            cnt = jnp.int32(0)
            for c in range(n_groups // 16):
                gv = gmin_v[pl.ds(c * 16, 16)]
                cnt = cnt + plsc.all_reduce_population_count(gv < mid)[0]
            return lax.cond(cnt >= k, lambda: (lo, mid), lambda: (mid, hi))

        lo0, hi0 = lax.fori_loop(
            0, 15, bs, (jnp.float32(-1.0), jnp.float32(4.0)))
        t_s[0] = hi0

        def gchunk(gc, _):
            gv = gmin_v[pl.ds(gc * 16, 16)]
            hit = gv < t_s[0]
            nh = plsc.all_reduce_population_count(hit)[0]

            @pl.when(nh > 0)
            def _():
                plsc.store_compressed(ghits.at[0:16], gc * 16 + lane, mask=hit)

                def scang(i, _):
                    g = plsc.load_gather(
                        ghits, [jnp.full((16,), i, jnp.int32)])[0]
                    base = g * gsz
                    for b in range(grp):
                        v = row_v[pl.ds(base + b * 16, 16)]
                        msk = v < t_s[0]
                        nn = plsc.all_reduce_population_count(msk)[0]

                        @pl.when(nn > 0)
                        def _():
                            c = cnt_s[0]
                            plsc.store_compressed(
                                bufd.at[pl.ds(c, 16)], v, mask=msk)
                            plsc.store_compressed(
                                bufi.at[pl.ds(c, 16)], base + b * 16 + lane,
                                mask=msk)
                            cnt_s[0] = c + nn

                            @pl.when(c + nn >= _TRIG)
                            def _():
                                distill()

                    return 0

                lax.fori_loop(0, nh, scang, 0)

            return 0

        lax.fori_loop(0, n_groups // 16, gchunk, 0)
        distill()
        pltpu.sync_copy(outi, idx_ref.at[row])
        pltpu.sync_copy(outd, d2o_ref.at[row])
        return 0

    lax.fori_loop(0, rows_per, do_row, 0)


def _sc_topk(d2, gmin, k, grp):
    import jax.experimental.pallas.tpu_sc as plsc

    m, n = d2.shape
    ng = n // (16 * grp)
    mesh = plsc.VectorSubcoreMesh(core_axis_name="c", subcore_axis_name="s")
    fn = pl.kernel(
        functools.partial(_sc_topk_body, m=m, n=n, k=k, grp=grp),
        out_type=(
            jax.ShapeDtypeStruct((m, k), jnp.int32),
            jax.ShapeDtypeStruct((m, k), jnp.float32),
        ),
        mesh=mesh,
        compiler_params=pltpu.CompilerParams(needs_layout_passes=False),
        scratch_types=[
            pltpu.VMEM((n,), jnp.float32),
            pltpu.VMEM((ng,), jnp.float32),
            pltpu.VMEM((16,), jnp.int32),
            pltpu.VMEM((_BUF,), jnp.float32),
            pltpu.VMEM((_BUF,), jnp.int32),
            pltpu.VMEM((k,), jnp.float32),
            pltpu.VMEM((k,), jnp.int32),
            pltpu.VMEM((16,), jnp.float32),
            pltpu.VMEM((16,), jnp.int32),
            pltpu.SMEM((1,), jnp.int32),
            pltpu.SMEM((1,), jnp.float32),
        ],
    )
    return fn(d2, gmin)


def _knn_radius(x, y, r, k, blk, grp):
    d2, gmin3 = _d2_matrix(y, x, blk, 16 * grp)
    gmin = gmin3.transpose(1, 0, 2).reshape(y.shape[0], -1)
    idx, d2sel = _sc_topk(d2, gmin, k, grp)
    valid = d2sel <= r * r
    return idx, valid


def kernel(points, batch, enc0_W, enc0_b, enc1_W, enc1_b, dec0_W, dec0_b, dec1_W, dec1_b):
    del batch
    fps1, p1 = _fps(points, M1, 8, N // 8)
    fps2, p2 = _fps(p1, M2, 8, M1 // 8)

    idx0, valid0 = _knn_radius(points, p1, R0, K, 2048, 8)
    rel0 = jnp.where(valid0[..., None], (points[idx0] - p1[:, None, :]) / R0, 0.0)
    h0 = jax.nn.relu(rel0.reshape(-1, 3) @ enc0_W + enc0_b)
    h0 = jnp.where(valid0.reshape(-1, 1), h0, 0.0)
    f1 = h0.reshape(M1, K, D0).max(axis=1)

    idx1, valid1 = _knn_radius(p1, p2, R1, K, 1024, 4)
    rel1 = jnp.where(valid1[..., None], (p1[idx1] - p2[:, None, :]) / R1, 0.0)
    g1 = jnp.where(valid1[..., None], f1[idx1], 0.0)
    h1 = jax.nn.relu(jnp.concatenate([rel1, g1], axis=-1).reshape(-1, 3 + D0) @ enc1_W + enc1_b)
    h1 = jnp.where(valid1.reshape(-1, 1), h1, 0.0)
    f2 = h1.reshape(M2, K, D1).max(axis=1)

    cur = idx1.reshape(-1)
    input_points1 = p1[cur]
    nxt = idx0[cur].reshape(-1)
    input_points0 = points[nxt]

    d0 = (f2 @ dec0_W + dec0_b).reshape(M2, K, 3 + D0)
    rel_a = jnp.tanh(d0[..., :3]).reshape(M2 * K, 3)
    feat_a = jax.nn.relu(d0[..., 3:]).reshape(M2 * K, D0)
    d1 = (feat_a @ dec1_W + dec1_b).reshape(M2 * K, K, 3)
    rel_b = jnp.tanh(d1)
    out1 = p2
    out2 = (out1[:, None, :] + rel_a.reshape(M2, K, 3) * R1).reshape(M2 * K, 3)
    out3 = (out2[:, None, :] + rel_b * R0).reshape(M2 * K * K, 3)
    return (out3, f2, input_points0, input_points1)


# speedup vs baseline: 2.6012x; 1.0587x over previous
import functools

import jax
import jax.numpy as jnp
from jax import lax
from jax.experimental import pallas as pl
from jax.experimental.pallas import tpu as pltpu

N = 32768
K = 32
R0 = 0.2
R1 = 0.5
M1 = N // K
M2 = M1 // K
D0 = 64
D1 = 128


def _fps_kernel(xs_ref, ys_ref, zs_ref, pr_ref, sel_ref, p1_ref, mind_ref, *, m, S, L):
    idx_plane = (lax.broadcasted_iota(jnp.int32, (S, L), 0) * L
                 + lax.broadcasted_iota(jnp.int32, (S, L), 1))
    big = jnp.int32(S * L)

    row0 = pr_ref[0:1, :]
    qx = row0[:, 0:1]
    qy = row0[:, 1:2]
    qz = row0[:, 2:3]
    mind_ref[...] = ((xs_ref[...] - qx) ** 2 + (ys_ref[...] - qy) ** 2
                     + (zs_ref[...] - qz) ** 2)
    sel_ref[0] = jnp.int32(0)
    p1_ref[0:1, :] = row0

    def body(i, _):
        mind = mind_ref[...]
        mx = jnp.max(mind)
        nxt = jnp.min(jnp.where(mind == mx, idx_plane, big))
        sel_ref[i] = nxt
        row = pr_ref[pl.ds(nxt, 1), :]
        p1_ref[pl.ds(i, 1), :] = row
        qx = row[:, 0:1]
        qy = row[:, 1:2]
        qz = row[:, 2:3]
        d2 = ((xs_ref[...] - qx) ** 2 + (ys_ref[...] - qy) ** 2
              + (zs_ref[...] - qz) ** 2)
        mind_ref[...] = jnp.minimum(mind, d2)
        return 0

    lax.fori_loop(1, m, body, 0)


def _fps(pts, m, S, L):
    n = pts.shape[0]
    planes = pts.T.reshape(3, S, L)
    sel, p_sel = pl.pallas_call(
        functools.partial(_fps_kernel, m=m, S=S, L=L),
        out_shape=(
            jax.ShapeDtypeStruct((m,), jnp.int32),
            jax.ShapeDtypeStruct((m, 3), jnp.float32),
        ),
        in_specs=[
            pl.BlockSpec(memory_space=pltpu.MemorySpace.VMEM),
            pl.BlockSpec(memory_space=pltpu.MemorySpace.VMEM),
            pl.BlockSpec(memory_space=pltpu.MemorySpace.VMEM),
            pl.BlockSpec(memory_space=pltpu.MemorySpace.VMEM),
        ],
        out_specs=(
            pl.BlockSpec(memory_space=pltpu.MemorySpace.SMEM),
            pl.BlockSpec(memory_space=pltpu.MemorySpace.VMEM),
        ),
        scratch_shapes=[pltpu.VMEM((S, L), jnp.float32)],
    )(planes[0], planes[1], planes[2], pts)
    return sel, p_sel


def _d2_block_kernel(y_ref, xt_ref, out_ref, gmin_ref, *, gsz):
    y = y_ref[...]
    xt = xt_ref[...]
    sy = jnp.sum(y * y, axis=1, keepdims=True)
    sx = jnp.sum(xt * xt, axis=0, keepdims=True)
    d2 = sy + sx - 2.0 * jnp.dot(y, xt)
    out_ref[...] = d2
    m, blk = d2.shape
    gmin_ref[...] = d2.reshape(m, blk // gsz, gsz).min(axis=-1)[None]


def _d2_matrix(y, x, blk, gsz):
    m = y.shape[0]
    n = x.shape[0]
    xt = x.T
    return pl.pallas_call(
        functools.partial(_d2_block_kernel, gsz=gsz),
        grid=(n // blk,),
        in_specs=[
            pl.BlockSpec((m, 3), lambda i: (0, 0)),
            pl.BlockSpec((3, blk), lambda i: (0, i)),
        ],
        out_specs=(
            pl.BlockSpec((m, blk), lambda i: (0, i)),
            pl.BlockSpec((1, m, blk // gsz), lambda i: (i, 0, 0)),
        ),
        out_shape=(
            jax.ShapeDtypeStruct((m, n), jnp.float32),
            jax.ShapeDtypeStruct((n // blk, m, blk // gsz), jnp.float32),
        ),
    )(y, xt)


_SC_NC = 2
_SC_NS = 16
_SC_NW = _SC_NC * _SC_NS

_BUF = 160
_TRIG = 128
_GRP = 8


def _sc_topk_body(d2_ref, gmin_ref, idx_ref, d2o_ref, row_v, gmin_v, ghits,
                  bufd, bufi, outd, outi, tmpf, tmpi, cnt_s, t_s,
                  *, m, n, k, grp):
    import jax.experimental.pallas.tpu_sc as plsc

    rows_per = m // _SC_NW
    nb = _BUF // 16
    gsz = 16 * grp
    n_groups = n // gsz
    wid = lax.axis_index("s") * _SC_NC + lax.axis_index("c")
    inf = jnp.float32(jnp.inf)
    bigi = jnp.int32(2**30)
    lane = lax.iota(jnp.int32, 16)
    inf_v = jnp.full((16,), inf, jnp.float32)
    bigi_v = jnp.full((16,), bigi, jnp.int32)

    def _smin_f(vec):
        for d in (8, 4, 2, 1):
            tmpf[0:16] = vec
            vec = jnp.minimum(vec, plsc.load_gather(tmpf, [lane ^ d]))
        return vec[0]

    def _smin_i(vec):
        for d in (8, 4, 2, 1):
            tmpi[0:16] = vec
            vec = jnp.minimum(vec, plsc.load_gather(tmpi, [lane ^ d]))
        return vec[0]

    def distill():
        rd0 = tuple(bufd[pl.ds(b * 16, 16)] for b in range(nb))
        ri0 = tuple(bufi[pl.ds(b * 16, 16)] for b in range(nb))

        def extract(t, st):
            o0d, o1d, o0i, o1i, _, rd, ri = st
            acc = rd[0]
            for b in range(1, nb):
                acc = jnp.minimum(acc, rd[b])
            mv = _smin_f(acc)
            ii = bigi_v
            for b in range(nb):
                ii = jnp.minimum(ii, jnp.where(rd[b] == mv, ri[b], bigi))
            mi = _smin_i(ii)
            rd = tuple(
                jnp.where((rd[b] == mv) & (ri[b] == mi), inf, rd[b])
                for b in range(nb))
            tl = jnp.where(t < 16, t, t - 16)
            le = lane == tl
            in0 = t < 16
            o0d = jnp.where(le & in0, mv, o0d)
            o1d = jnp.where(le & (~in0), mv, o1d)
            o0i = jnp.where(le & in0, mi, o0i)
            o1i = jnp.where(le & (~in0), mi, o1i)
            return (o0d, o1d, o0i, o1i, mv, rd, ri)

        o0d, o1d, o0i, o1i, lmv, _, _ = lax.fori_loop(
            0, k, extract, (inf_v, inf_v, bigi_v, bigi_v, inf, rd0, ri0))
        outd[0:16] = o0d
        outd[pl.ds(16, 16)] = o1d
        outi[0:16] = o0i
        outi[pl.ds(16, 16)] = o1i
        bufd[0:16] = o0d
        bufd[pl.ds(16, 16)] = o1d
        bufi[0:16] = o0i
        bufi[pl.ds(16, 16)] = o1i
        for b in range(2, nb):
            bufd[pl.ds(b * 16, 16)] = inf_v
            bufi[pl.ds(b * 16, 16)] = bigi_v
        cnt_s[0] = jnp.int32(k)
        t_s[0] = lmv

    def do_row(j, _):
        row = wid * rows_per + j
        pltpu.sync_copy(d2_ref.at[row], row_v)
        pltpu.sync_copy(gmin_ref.at[row], gmin_v)
        for b in range(nb):
            bufd[pl.ds(b * 16, 16)] = inf_v
            bufi[pl.ds(b * 16, 16)] = bigi_v
        cnt_s[0] = jnp.int32(0)

        def bs(i, st):
            lo, hi = st
            mid = 0.5 * (lo + hi)
            cnt = jnp.int32(0)
            for c in range(n_groups // 16):
                gv = gmin_v[pl.ds(c * 16, 16)]
                cnt = cnt + plsc.all_reduce_population_count(gv < mid)[0]
            return lax.cond(cnt >= k, lambda: (lo, mid), lambda: (mid, hi))

        lo0, hi0 = lax.fori_loop(
            0, 15, bs, (jnp.float32(-1.0), jnp.float32(4.0)))
        t_s[0] = hi0

        def gchunk(gc, _):
            gv = gmin_v[pl.ds(gc * 16, 16)]
            hit = gv < t_s[0]
            nh = plsc.all_reduce_population_count(hit)[0]

            @pl.when(nh > 0)
            def _():
                plsc.store_compressed(ghits.at[0:16], gc * 16 + lane, mask=hit)

                def scang(i, _):
                    g = plsc.load_gather(
                        ghits, [jnp.full((16,), i, jnp.int32)])[0]
                    base = g * gsz
                    for b in range(grp):
                        v = row_v[pl.ds(base + b * 16, 16)]
                        msk = v < t_s[0]
                        nn = plsc.all_reduce_population_count(msk)[0]

                        @pl.when(nn > 0)
                        def _():
                            c = cnt_s[0]
                            plsc.store_compressed(
                                bufd.at[pl.ds(c, 16)], v, mask=msk)
                            plsc.store_compressed(
                                bufi.at[pl.ds(c, 16)], base + b * 16 + lane,
                                mask=msk)
                            cnt_s[0] = c + nn

                            @pl.when(c + nn >= _TRIG)
                            def _():
                                distill()

                    return 0

                lax.fori_loop(0, nh, scang, 0)

            return 0

        lax.fori_loop(0, n_groups // 16, gchunk, 0)
        distill()
        pltpu.sync_copy(outi, idx_ref.at[row])
        pltpu.sync_copy(outd, d2o_ref.at[row])
        return 0

    lax.fori_loop(0, rows_per, do_row, 0)


def _sc_topk(d2, gmin, k, grp):
    import jax.experimental.pallas.tpu_sc as plsc

    m, n = d2.shape
    ng = n // (16 * grp)
    mesh = plsc.VectorSubcoreMesh(core_axis_name="c", subcore_axis_name="s")
    fn = pl.kernel(
        functools.partial(_sc_topk_body, m=m, n=n, k=k, grp=grp),
        out_type=(
            jax.ShapeDtypeStruct((m, k), jnp.int32),
            jax.ShapeDtypeStruct((m, k), jnp.float32),
        ),
        mesh=mesh,
        compiler_params=pltpu.CompilerParams(needs_layout_passes=False),
        scratch_types=[
            pltpu.VMEM((n,), jnp.float32),
            pltpu.VMEM((ng,), jnp.float32),
            pltpu.VMEM((16,), jnp.int32),
            pltpu.VMEM((_BUF,), jnp.float32),
            pltpu.VMEM((_BUF,), jnp.int32),
            pltpu.VMEM((k,), jnp.float32),
            pltpu.VMEM((k,), jnp.int32),
            pltpu.VMEM((16,), jnp.float32),
            pltpu.VMEM((16,), jnp.int32),
            pltpu.SMEM((1,), jnp.int32),
            pltpu.SMEM((1,), jnp.float32),
        ],
    )
    return fn(d2, gmin)


def _knn_radius(x, y, r, k, blk, grp):
    d2, gmin3 = _d2_matrix(y, x, blk, 16 * grp)
    gmin = gmin3.transpose(1, 0, 2).reshape(y.shape[0], -1)
    idx, d2sel = _sc_topk(d2, gmin, k, grp)
    valid = d2sel <= r * r
    return idx, valid


def kernel(points, batch, enc0_W, enc0_b, enc1_W, enc1_b, dec0_W, dec0_b, dec1_W, dec1_b):
    del batch
    fps1, p1 = _fps(points, M1, 8, N // 8)
    fps2, p2 = _fps(p1, M2, 8, M1 // 8)

    idx0, valid0 = _knn_radius(points, p1, R0, K, 2048, 8)
    rel0 = jnp.where(valid0[..., None], (points[idx0] - p1[:, None, :]) / R0, 0.0)
    h0 = jax.nn.relu(rel0.reshape(-1, 3) @ enc0_W + enc0_b)
    h0 = jnp.where(valid0.reshape(-1, 1), h0, 0.0)
    f1 = h0.reshape(M1, K, D0).max(axis=1)

    idx1, valid1 = _knn_radius(p1, p2, R1, K, 1024, 4)
    rel1 = jnp.where(valid1[..., None], (p1[idx1] - p2[:, None, :]) / R1, 0.0)
    g1 = jnp.where(valid1[..., None], f1[idx1], 0.0)
    h1 = jax.nn.relu(jnp.concatenate([rel1, g1], axis=-1).reshape(-1, 3 + D0) @ enc1_W + enc1_b)
    h1 = jnp.where(valid1.reshape(-1, 1), h1, 0.0)
    f2 = h1.reshape(M2, K, D1).max(axis=1)

    cur = idx1.reshape(-1)
    input_points1 = p1[cur]
    nxt = idx0[cur].reshape(-1)
    input_points0 = points[nxt]

    d0 = (f2 @ dec0_W + dec0_b).reshape(M2, K, 3 + D0)
    rel_a = jnp.tanh(d0[..., :3]).reshape(M2 * K, 3)
    feat_a = jax.nn.relu(d0[..., 3:]).reshape(M2 * K, D0)
    d1 = (feat_a @ dec1_W + dec1_b).reshape(M2 * K, K, 3)
    rel_b = jnp.tanh(d1)
    out1 = p2
    out2 = (out1[:, None, :] + rel_a.reshape(M2, K, 3) * R1).reshape(M2 * K, 3)
    out3 = (out2[:, None, :] + rel_b * R0).reshape(M2 * K * K, 3)
    return (out3, f2, input_points0, input_points1)
